# baseline (speedup 1.0000x reference)
import functools

import jax
import jax.numpy as jnp
from jax import lax
from jax.experimental import pallas as pl
from jax.experimental.pallas import tpu as pltpu
from jax.experimental.pallas import tpu_sc as plsc

N_NODES = 50000
N_EDGES = 800000
IN_DIM = 128
HID = 64
HALF = HID // 2
OUT_DIM = 60
NHALF = N_NODES // 2

NC = 2
NS = 16

CHUNK = 128
GROUP = 25
NGROUP = 16
NRING = 4
AHEAD = NRING - 1
EDGES_PER_TILE = CHUNK * GROUP * NGROUP
E_PAD = EDGES_PER_TILE * NS
NCHUNKS = E_PAD // CHUNK
CH_PER_TILE = NCHUNKS // NS

REGION = 25088
AGG_ROWS = 2 * REGION
AGG_ROWS_PER_TILE = AGG_ROWS // NS
TRASH = REGION - NHALF
OUT_ROWS = 25024
OUT_ROWS_PER_TILE = OUT_ROWS // NS

ROW_BLK = 5000
GRID = NHALF // ROW_BLK


def _seg_sum_body(h2v, idx2, dstc, zfull, out, idx_v, dst_v, rows_v, agg_sh,
                  sem_g, sem_s, sem_i):
  c = lax.axis_index("c")
  s = lax.axis_index("s")
  ch_base = s * CH_PER_TILE

  def load_idx(g, buf):
    ch0 = ch_base + g * GROUP
    pltpu.async_copy(idx2.at[c, pl.ds(ch0, GROUP)], idx_v.at[buf], sem_i)
    pltpu.async_copy(dstc.at[pl.ds(ch0, GROUP)], dst_v.at[buf], sem_i)

  def wait_idx():
    pltpu.make_async_copy(idx2.at[c, pl.ds(ch_base, GROUP)],
                          idx_v.at[0], sem_i).wait()
    pltpu.make_async_copy(dstc.at[pl.ds(ch_base, GROUP)],
                          dst_v.at[0], sem_i).wait()

  def fire_g(gb, j, slot):
    pltpu.async_copy(h2v.at[idx_v.at[gb, j]], rows_v.at[slot], sem_g)

  def wait_g(slot):
    pltpu.make_async_copy(h2v.at[idx_v.at[0, 0]], rows_v.at[slot],
                          sem_g).wait()

  def fire_s(gb, j, slot):
    pltpu.async_copy(rows_v.at[slot], agg_sh.at[dst_v.at[gb, j]], sem_s,
                     add=True)

  def wait_s(slot):
    pltpu.make_async_copy(rows_v.at[slot], agg_sh.at[dst_v.at[0, 0]],
                          sem_s).wait()

  load_idx(jnp.int32(0), jnp.int32(0))
  pltpu.sync_copy(zfull.at[pl.ds(s * AGG_ROWS_PER_TILE, AGG_ROWS_PER_TILE)],
                  agg_sh.at[pl.ds(s * AGG_ROWS_PER_TILE, AGG_ROWS_PER_TILE)])
  plsc.subcore_barrier()
  wait_idx()

  def group_body(g, carry):
    gb = lax.rem(g, 2)
    ng = lax.min(g + 1, NGROUP - 1)
    load_idx(ng, 1 - gb)

    for j in range(AHEAD):
      fire_g(gb, j, j)
    wait_g(0)
    fire_s(gb, 0, 0)
    fire_g(gb, AHEAD, AHEAD)

    def chunk_body(j, carry2):
      slot = lax.rem(j, NRING)
      wait_g(slot)
      fire_s(gb, j, slot)
      wait_s(lax.rem(j - 1, NRING))
      fire_g(gb, j + AHEAD, lax.rem(j + AHEAD, NRING))
      return carry2

    lax.fori_loop(1, GROUP - AHEAD, chunk_body, 0, unroll=3)
    for j in range(GROUP - AHEAD, GROUP):
      slot = j % NRING
      wait_g(slot)
      fire_s(gb, j, slot)
      wait_s((j - 1) % NRING)
    wait_s((GROUP - 1) % NRING)
    wait_idx()
    return carry

  lax.fori_loop(0, NGROUP, group_body, 0)
  plsc.subcore_barrier()

  base = s * OUT_ROWS_PER_TILE
  pltpu.sync_copy(
      agg_sh.at[pl.ds(base, OUT_ROWS_PER_TILE)],
      out.at[pl.ds(base, OUT_ROWS_PER_TILE), pl.ds(c * HALF, HALF)])
  pltpu.sync_copy(
      agg_sh.at[pl.ds(REGION + base, OUT_ROWS_PER_TILE)],
      out.at[pl.ds(base, OUT_ROWS_PER_TILE), pl.ds(HID + c * HALF, HALF)])


_seg_sum = functools.partial(
    pl.kernel,
    out_type=jax.ShapeDtypeStruct((OUT_ROWS, 2 * HID), jnp.float32),
    mesh=plsc.VectorSubcoreMesh(core_axis_name="c", subcore_axis_name="s"),
    scratch_types=[
        pltpu.VMEM((2, GROUP, CHUNK), jnp.int32),
        pltpu.VMEM((2, GROUP, CHUNK), jnp.int32),
        pltpu.VMEM((NRING, CHUNK, HALF), jnp.float32),
        pltpu.VMEM_SHARED((AGG_ROWS, HALF), jnp.float32),
        pltpu.SemaphoreType.DMA,
        pltpu.SemaphoreType.DMA,
        pltpu.SemaphoreType.DMA,
    ],
    compiler_params=pltpu.CompilerParams(use_tc_tiling_on_sc=False),
)(_seg_sum_body)


def _mlp_body(x3, w1, b1, w2, b2, w3, b3, w4, b4, out):
  xx = jnp.concatenate([x3[0], x3[1]], axis=0)
  h = jnp.maximum(jnp.dot(xx, w1[...], preferred_element_type=jnp.float32)
                  + b1[...], 0.0)
  h = jnp.maximum(jnp.dot(h, w2[...], preferred_element_type=jnp.float32)
                  + b2[...], 0.0)
  h = jnp.maximum(jnp.dot(h, w3[...], preferred_element_type=jnp.float32)
                  + b3[...], 0.0) + h
  h = jnp.maximum(jnp.dot(h, w4[...], preferred_element_type=jnp.float32)
                  + b4[...], 0.0) + h
  out[...] = jnp.concatenate([h[:ROW_BLK], h[ROW_BLK:]], axis=1)


def _full(shape):
  return pl.BlockSpec(shape, lambda i: (0,) * len(shape))


def _mlp(x3, w1t, b1, w2t, b2, w3t, b3, w4t, b4):
  return pl.pallas_call(
      _mlp_body,
      grid=(GRID,),
      in_specs=[
          pl.BlockSpec((2, ROW_BLK, IN_DIM), lambda i: (0, i, 0)),
          _full((IN_DIM, HID)), _full((1, HID)),
          _full((HID, HID)), _full((1, HID)),
          _full((HID, HID)), _full((1, HID)),
          _full((HID, HID)), _full((1, HID)),
      ],
      out_specs=pl.BlockSpec((ROW_BLK, 2 * HID), lambda i: (i, 0)),
      out_shape=jax.ShapeDtypeStruct((NHALF, 2 * HID), jnp.float32),
  )(x3, w1t, b1, w2t, b2, w3t, b3, w4t, b4)


def _conv_combine_body(aggp, hp, wr, br, wroot, out):
  g = jnp.dot(aggp[...], wr[...], preferred_element_type=jnp.float32)
  g += jnp.dot(hp[...], wroot[...], preferred_element_type=jnp.float32)
  out[...] = jnp.maximum(g + br[...], 0.0)


def _conv_combine(aggp, hp, wr_bd, br_bd, wroot_bd):
  return pl.pallas_call(
      _conv_combine_body,
      grid=(GRID,),
      in_specs=[
          pl.BlockSpec((ROW_BLK, 2 * HID), lambda i: (i, 0)),
          pl.BlockSpec((ROW_BLK, 2 * HID), lambda i: (i, 0)),
          _full((2 * HID, 2 * HID)), _full((1, 2 * HID)),
          _full((2 * HID, 2 * HID)),
      ],
      out_specs=pl.BlockSpec((ROW_BLK, 2 * HID), lambda i: (i, 0)),
      out_shape=jax.ShapeDtypeStruct((NHALF, 2 * HID), jnp.float32),
  )(aggp, hp, wr_bd, br_bd, wroot_bd)


def _final_body(aggp, hp, wr, br, wroot, w5, b5, out):
  g = jnp.dot(aggp[...], wr[...], preferred_element_type=jnp.float32)
  g += jnp.dot(hp[...], wroot[...], preferred_element_type=jnp.float32)
  h2 = jnp.maximum(g + br[...], 0.0)
  out[...] = jnp.dot(h2, w5[...], preferred_element_type=jnp.float32) + b5[...]


def _final(aggp, hp, wr_bd, br_bd, wroot_bd, w5_bd, b5_bd):
  return pl.pallas_call(
      _final_body,
      grid=(GRID,),
      in_specs=[
          pl.BlockSpec((ROW_BLK, 2 * HID), lambda i: (i, 0)),
          pl.BlockSpec((ROW_BLK, 2 * HID), lambda i: (i, 0)),
          _full((2 * HID, 2 * HID)), _full((1, 2 * HID)),
          _full((2 * HID, 2 * HID)),
          _full((2 * HID, 2 * OUT_DIM)), _full((1, 2 * OUT_DIM)),
      ],
      out_specs=pl.BlockSpec((ROW_BLK, 2 * OUT_DIM), lambda i: (i, 0)),
      out_shape=jax.ShapeDtypeStruct((NHALF, 2 * OUT_DIM), jnp.float32),
  )(aggp, hp, wr_bd, br_bd, wroot_bd, w5_bd, b5_bd)


def _blockdiag(a):
  n, m = a.shape
  z = jnp.zeros((n, m), a.dtype)
  return jnp.concatenate([
      jnp.concatenate([a, z], axis=1),
      jnp.concatenate([z, a], axis=1)], axis=0)


def kernel(x, edge_index, W1, b1, W2, b2, W3, b3, W4, b4,
           Wrel1, brel1, Wroot1, Wrel2, brel2, Wroot2, W5, b5):
  src = edge_index[0].astype(jnp.int32)
  dst = edge_index[1].astype(jnp.int32)

  npad = E_PAD - N_EDGES
  i_pad = jnp.arange(npad, dtype=jnp.int32)
  srcp = jnp.concatenate([src, i_pad % N_NODES])
  src4 = jnp.where(srcp < NHALF, 4 * srcp, 4 * (srcp - NHALF) + 2)
  idx2 = jnp.stack([src4, src4 + 1]).reshape(NC, NCHUNKS, CHUNK)
  dst2 = jnp.where(dst < NHALF, dst, dst + TRASH)
  pad_dst = jnp.where(i_pad % 2 == 0,
                      NHALF + (i_pad // 2) % TRASH,
                      REGION + NHALF + (i_pad // 2) % TRASH)
  dstc = jnp.concatenate([dst2, pad_dst]).reshape(NCHUNKS, CHUNK)
  zfull = jnp.zeros((AGG_ROWS, HALF), jnp.float32)

  x3 = x.reshape(2, NHALF, IN_DIM)
  hp = _mlp(x3, W1.T, b1.reshape(1, HID), W2.T, b2.reshape(1, HID),
            W3.T, b3.reshape(1, HID), W4.T, b4.reshape(1, HID))

  wrel1_bd = _blockdiag(Wrel1.T)
  wroot1_bd = _blockdiag(Wroot1.T)
  brel1_bd = jnp.concatenate([brel1, brel1]).reshape(1, 2 * HID)
  wrel2_bd = _blockdiag(Wrel2.T)
  wroot2_bd = _blockdiag(Wroot2.T)
  brel2_bd = jnp.concatenate([brel2, brel2]).reshape(1, 2 * HID)
  w5_bd = _blockdiag(W5.T)
  b5_bd = jnp.concatenate([b5, b5]).reshape(1, 2 * OUT_DIM)

  agg1 = _seg_sum(hp.reshape(4 * NHALF, HALF), idx2, dstc, zfull)
  h1p = _conv_combine(agg1, hp, wrel1_bd, brel1_bd, wroot1_bd)

  agg2 = _seg_sum(h1p.reshape(4 * NHALF, HALF), idx2, dstc, zfull)
  outp = _final(agg2, h1p, wrel2_bd, brel2_bd, wroot2_bd,
                w5_bd, b5_bd)
  return jnp.concatenate([outp[:, :OUT_DIM], outp[:, OUT_DIM:]], axis=0)


# speedup vs baseline: 1.0435x; 1.0435x over previous
import functools

import jax
import jax.numpy as jnp
from jax import lax
from jax.experimental import pallas as pl
from jax.experimental.pallas import tpu as pltpu
from jax.experimental.pallas import tpu_sc as plsc

N_NODES = 50000
N_EDGES = 800000
IN_DIM = 128
HID = 64
HALF = HID // 2
OUT_DIM = 60
NHALF = N_NODES // 2

NC = 2
NS = 16

CHUNK = 128
GROUP = 16
NGROUP = 25
NRING = 5
AHEAD = NRING - 1
EDGES_PER_TILE = CHUNK * GROUP * NGROUP
E_PAD = EDGES_PER_TILE * NS
NCHUNKS = E_PAD // CHUNK
CH_PER_TILE = NCHUNKS // NS

REGION = 25088
AGG_ROWS = 2 * REGION
AGG_ROWS_PER_TILE = AGG_ROWS // NS
TRASH = REGION - NHALF
OUT_ROWS = 25024
OUT_ROWS_PER_TILE = OUT_ROWS // NS

ROW_BLK = 5000
GRID = NHALF // ROW_BLK


def _seg_sum_body(h2v, idx2, dstc, zfull, out, idx_v, dst_v, rows_v, agg_sh,
                  sem_g, sem_s, sem_i):
  c = lax.axis_index("c")
  s = lax.axis_index("s")
  ch_base = s * CH_PER_TILE

  def load_idx(g, buf):
    ch0 = ch_base + g * GROUP
    pltpu.async_copy(idx2.at[c, pl.ds(ch0, GROUP)], idx_v.at[buf], sem_i)
    pltpu.async_copy(dstc.at[pl.ds(ch0, GROUP)], dst_v.at[buf], sem_i)

  def wait_idx():
    pltpu.make_async_copy(idx2.at[c, pl.ds(ch_base, GROUP)],
                          idx_v.at[0], sem_i).wait()
    pltpu.make_async_copy(dstc.at[pl.ds(ch_base, GROUP)],
                          dst_v.at[0], sem_i).wait()

  def fire_g(gb, j, slot):
    pltpu.async_copy(h2v.at[idx_v.at[gb, j]], rows_v.at[slot], sem_g)

  def wait_g(slot):
    pltpu.make_async_copy(h2v.at[idx_v.at[0, 0]], rows_v.at[slot],
                          sem_g).wait()

  def fire_s(gb, j, slot):
    pltpu.async_copy(rows_v.at[slot], agg_sh.at[dst_v.at[gb, j]], sem_s,
                     add=True)

  def wait_s(slot):
    pltpu.make_async_copy(rows_v.at[slot], agg_sh.at[dst_v.at[0, 0]],
                          sem_s).wait()

  load_idx(jnp.int32(0), jnp.int32(0))
  pltpu.sync_copy(zfull.at[pl.ds(s * AGG_ROWS_PER_TILE, AGG_ROWS_PER_TILE)],
                  agg_sh.at[pl.ds(s * AGG_ROWS_PER_TILE, AGG_ROWS_PER_TILE)])
  plsc.subcore_barrier()
  wait_idx()

  def group_body(g, carry):
    gb = lax.rem(g, 2)
    ng = lax.min(g + 1, NGROUP - 1)
    load_idx(ng, 1 - gb)

    for j in range(AHEAD):
      fire_g(gb, j, j)
    wait_g(0)
    fire_s(gb, 0, 0)
    fire_g(gb, AHEAD, AHEAD)

    def chunk_body(j, carry2):
      slot = lax.rem(j, NRING)
      wait_g(slot)
      fire_s(gb, j, slot)
      wait_s(lax.rem(j - 1, NRING))
      fire_g(gb, j + AHEAD, lax.rem(j + AHEAD, NRING))
      return carry2

    lax.fori_loop(1, GROUP - AHEAD, chunk_body, 0, unroll=3)
    for j in range(GROUP - AHEAD, GROUP):
      slot = j % NRING
      wait_g(slot)
      fire_s(gb, j, slot)
      wait_s((j - 1) % NRING)
    wait_s((GROUP - 1) % NRING)
    wait_idx()
    return carry

  lax.fori_loop(0, NGROUP, group_body, 0)
  plsc.subcore_barrier()

  base = s * OUT_ROWS_PER_TILE
  pltpu.sync_copy(
      agg_sh.at[pl.ds(base, OUT_ROWS_PER_TILE)],
      out.at[pl.ds(base, OUT_ROWS_PER_TILE), pl.ds(c * HALF, HALF)])
  pltpu.sync_copy(
      agg_sh.at[pl.ds(REGION + base, OUT_ROWS_PER_TILE)],
      out.at[pl.ds(base, OUT_ROWS_PER_TILE), pl.ds(HID + c * HALF, HALF)])


_seg_sum = functools.partial(
    pl.kernel,
    out_type=jax.ShapeDtypeStruct((OUT_ROWS, 2 * HID), jnp.float32),
    mesh=plsc.VectorSubcoreMesh(core_axis_name="c", subcore_axis_name="s"),
    scratch_types=[
        pltpu.VMEM((2, GROUP, CHUNK), jnp.int32),
        pltpu.VMEM((2, GROUP, CHUNK), jnp.int32),
        pltpu.VMEM((NRING, CHUNK, HALF), jnp.float32),
        pltpu.VMEM_SHARED((AGG_ROWS, HALF), jnp.float32),
        pltpu.SemaphoreType.DMA,
        pltpu.SemaphoreType.DMA,
        pltpu.SemaphoreType.DMA,
    ],
    compiler_params=pltpu.CompilerParams(use_tc_tiling_on_sc=False),
)(_seg_sum_body)


def _mlp_body(x3, w1, b1, w2, b2, w3, b3, w4, b4, out):
  xx = jnp.concatenate([x3[0], x3[1]], axis=0)
  h = jnp.maximum(jnp.dot(xx, w1[...], preferred_element_type=jnp.float32)
                  + b1[...], 0.0)
  h = jnp.maximum(jnp.dot(h, w2[...], preferred_element_type=jnp.float32)
                  + b2[...], 0.0)
  h = jnp.maximum(jnp.dot(h, w3[...], preferred_element_type=jnp.float32)
                  + b3[...], 0.0) + h
  h = jnp.maximum(jnp.dot(h, w4[...], preferred_element_type=jnp.float32)
                  + b4[...], 0.0) + h
  out[...] = jnp.concatenate([h[:ROW_BLK], h[ROW_BLK:]], axis=1)


def _full(shape):
  return pl.BlockSpec(shape, lambda i: (0,) * len(shape))


def _mlp(x3, w1t, b1, w2t, b2, w3t, b3, w4t, b4):
  return pl.pallas_call(
      _mlp_body,
      grid=(GRID,),
      in_specs=[
          pl.BlockSpec((2, ROW_BLK, IN_DIM), lambda i: (0, i, 0)),
          _full((IN_DIM, HID)), _full((1, HID)),
          _full((HID, HID)), _full((1, HID)),
          _full((HID, HID)), _full((1, HID)),
          _full((HID, HID)), _full((1, HID)),
      ],
      out_specs=pl.BlockSpec((ROW_BLK, 2 * HID), lambda i: (i, 0)),
      out_shape=jax.ShapeDtypeStruct((NHALF, 2 * HID), jnp.float32),
  )(x3, w1t, b1, w2t, b2, w3t, b3, w4t, b4)


def _conv_combine_body(aggp, hp, wr, br, wroot, out):
  g = jnp.dot(aggp[...], wr[...], preferred_element_type=jnp.float32)
  g += jnp.dot(hp[...], wroot[...], preferred_element_type=jnp.float32)
  out[...] = jnp.maximum(g + br[...], 0.0)


def _conv_combine(aggp, hp, wr_bd, br_bd, wroot_bd):
  return pl.pallas_call(
      _conv_combine_body,
      grid=(GRID,),
      in_specs=[
          pl.BlockSpec((ROW_BLK, 2 * HID), lambda i: (i, 0)),
          pl.BlockSpec((ROW_BLK, 2 * HID), lambda i: (i, 0)),
          _full((2 * HID, 2 * HID)), _full((1, 2 * HID)),
          _full((2 * HID, 2 * HID)),
      ],
      out_specs=pl.BlockSpec((ROW_BLK, 2 * HID), lambda i: (i, 0)),
      out_shape=jax.ShapeDtypeStruct((NHALF, 2 * HID), jnp.float32),
  )(aggp, hp, wr_bd, br_bd, wroot_bd)


def _final_body(aggp, hp, wr, br, wroot, w5, b5, out):
  g = jnp.dot(aggp[...], wr[...], preferred_element_type=jnp.float32)
  g += jnp.dot(hp[...], wroot[...], preferred_element_type=jnp.float32)
  h2 = jnp.maximum(g + br[...], 0.0)
  out[...] = jnp.dot(h2, w5[...], preferred_element_type=jnp.float32) + b5[...]


def _final(aggp, hp, wr_bd, br_bd, wroot_bd, w5_bd, b5_bd):
  return pl.pallas_call(
      _final_body,
      grid=(GRID,),
      in_specs=[
          pl.BlockSpec((ROW_BLK, 2 * HID), lambda i: (i, 0)),
          pl.BlockSpec((ROW_BLK, 2 * HID), lambda i: (i, 0)),
          _full((2 * HID, 2 * HID)), _full((1, 2 * HID)),
          _full((2 * HID, 2 * HID)),
          _full((2 * HID, 2 * OUT_DIM)), _full((1, 2 * OUT_DIM)),
      ],
      out_specs=pl.BlockSpec((ROW_BLK, 2 * OUT_DIM), lambda i: (i, 0)),
      out_shape=jax.ShapeDtypeStruct((NHALF, 2 * OUT_DIM), jnp.float32),
  )(aggp, hp, wr_bd, br_bd, wroot_bd, w5_bd, b5_bd)


def _blockdiag(a):
  n, m = a.shape
  z = jnp.zeros((n, m), a.dtype)
  return jnp.concatenate([
      jnp.concatenate([a, z], axis=1),
      jnp.concatenate([z, a], axis=1)], axis=0)


def kernel(x, edge_index, W1, b1, W2, b2, W3, b3, W4, b4,
           Wrel1, brel1, Wroot1, Wrel2, brel2, Wroot2, W5, b5):
  src = edge_index[0].astype(jnp.int32)
  dst = edge_index[1].astype(jnp.int32)

  npad = E_PAD - N_EDGES
  i_pad = jnp.arange(npad, dtype=jnp.int32)
  srcp = jnp.concatenate([src, i_pad % N_NODES])
  src4 = jnp.where(srcp < NHALF, 4 * srcp, 4 * (srcp - NHALF) + 2)
  idx2 = jnp.stack([src4, src4 + 1]).reshape(NC, NCHUNKS, CHUNK)
  dst2 = jnp.where(dst < NHALF, dst, dst + TRASH)
  pad_dst = jnp.where(i_pad % 2 == 0,
                      NHALF + (i_pad // 2) % TRASH,
                      REGION + NHALF + (i_pad // 2) % TRASH)
  dstc = jnp.concatenate([dst2, pad_dst]).reshape(NCHUNKS, CHUNK)
  zfull = jnp.zeros((AGG_ROWS, HALF), jnp.float32)

  x3 = x.reshape(2, NHALF, IN_DIM)
  hp = _mlp(x3, W1.T, b1.reshape(1, HID), W2.T, b2.reshape(1, HID),
            W3.T, b3.reshape(1, HID), W4.T, b4.reshape(1, HID))

  wrel1_bd = _blockdiag(Wrel1.T)
  wroot1_bd = _blockdiag(Wroot1.T)
  brel1_bd = jnp.concatenate([brel1, brel1]).reshape(1, 2 * HID)
  wrel2_bd = _blockdiag(Wrel2.T)
  wroot2_bd = _blockdiag(Wroot2.T)
  brel2_bd = jnp.concatenate([brel2, brel2]).reshape(1, 2 * HID)
  w5_bd = _blockdiag(W5.T)
  b5_bd = jnp.concatenate([b5, b5]).reshape(1, 2 * OUT_DIM)

  agg1 = _seg_sum(hp.reshape(4 * NHALF, HALF), idx2, dstc, zfull)
  h1p = _conv_combine(agg1, hp, wrel1_bd, brel1_bd, wroot1_bd)

  agg2 = _seg_sum(h1p.reshape(4 * NHALF, HALF), idx2, dstc, zfull)
  outp = _final(agg2, h1p, wrel2_bd, brel2_bd, wroot2_bd,
                w5_bd, b5_bd)
  return jnp.concatenate([outp[:, :OUT_DIM], outp[:, OUT_DIM:]], axis=0)


# speedup vs baseline: 1.1241x; 1.0772x over previous
import functools

import jax
import jax.numpy as jnp
from jax import lax
from jax.experimental import pallas as pl
from jax.experimental.pallas import tpu as pltpu
from jax.experimental.pallas import tpu_sc as plsc

N_NODES = 50000
N_EDGES = 800000
IN_DIM = 128
HID = 64
HALF = HID // 2
OUT_DIM = 60
NHALF = N_NODES // 2

NC = 2
NS = 16

CHUNK = 128
GROUP = 16
NGROUP = 25
NRING = 5
AHEAD = NRING - 1
EDGES_PER_TILE = CHUNK * GROUP * NGROUP
E_PAD = EDGES_PER_TILE * NS
NCHUNKS = E_PAD // CHUNK
CH_PER_TILE = NCHUNKS // NS

REGION = 25088
AGG_ROWS = 2 * REGION
AGG_ROWS_PER_TILE = AGG_ROWS // NS
TRASH = REGION - NHALF
OUT_ROWS = 25024
OUT_ROWS_PER_TILE = OUT_ROWS // NS

ROW_BLK = 5000
GRID = NHALF // ROW_BLK


def _seg_sum_body(h2v, idx2, dstc, zfull, out, idx_v, dst_v, rows_v, agg_sh,
                  sem_g, sem_s, sem_i):
  c = lax.axis_index("c")
  s = lax.axis_index("s")
  ch_base = s * CH_PER_TILE

  def load_idx(g, buf):
    ch0 = ch_base + g * GROUP
    pltpu.async_copy(idx2.at[c, pl.ds(ch0, GROUP)], idx_v.at[buf], sem_i)
    pltpu.async_copy(dstc.at[pl.ds(ch0, GROUP)], dst_v.at[buf], sem_i)

  def wait_idx():
    pltpu.make_async_copy(idx2.at[c, pl.ds(ch_base, GROUP)],
                          idx_v.at[0], sem_i).wait()
    pltpu.make_async_copy(dstc.at[pl.ds(ch_base, GROUP)],
                          dst_v.at[0], sem_i).wait()

  def fire_g(gb, j, slot):
    pltpu.async_copy(h2v.at[idx_v.at[gb, j]], rows_v.at[slot], sem_g)

  def wait_g(slot):
    pltpu.make_async_copy(h2v.at[idx_v.at[0, 0]], rows_v.at[slot],
                          sem_g).wait()

  def fire_s(gb, j, slot):
    pltpu.async_copy(rows_v.at[slot], agg_sh.at[dst_v.at[gb, j]], sem_s,
                     add=True)

  def wait_s(slot):
    pltpu.make_async_copy(rows_v.at[slot], agg_sh.at[dst_v.at[0, 0]],
                          sem_s).wait()

  load_idx(jnp.int32(0), jnp.int32(0))
  pltpu.sync_copy(zfull.at[pl.ds(s * AGG_ROWS_PER_TILE, AGG_ROWS_PER_TILE)],
                  agg_sh.at[pl.ds(s * AGG_ROWS_PER_TILE, AGG_ROWS_PER_TILE)])
  plsc.subcore_barrier()
  wait_idx()
  load_idx(jnp.int32(1), jnp.int32(1))

  z = jnp.int32(0)
  for k in range(AHEAD):
    fire_g(z, k, k)
  wait_g(0)
  fire_s(z, 0, 0)
  fire_g(z, AHEAD, AHEAD)

  def chunk_body(ch, carry):
    g = lax.div(ch, GROUP)
    j = lax.rem(ch, GROUP)
    slot = lax.rem(ch, NRING)
    wait_g(slot)
    fire_s(lax.rem(g, 2), j, slot)
    wait_s(slot)

    @pl.when(j == 0)
    def _load():
      load_idx(lax.min(g + 1, NGROUP - 1), lax.rem(g + 1, 2))

    @pl.when(j == GROUP - AHEAD)
    def _wait():
      wait_idx()

    chn = ch + AHEAD
    gn = lax.div(chn, GROUP)
    fire_g(lax.rem(gn, 2), lax.rem(chn, GROUP), lax.rem(chn, NRING))
    return carry

  lax.fori_loop(1, CH_PER_TILE - AHEAD, chunk_body, 0)
  lgb = jnp.int32((NGROUP - 1) % 2)
  for ch in range(CH_PER_TILE - AHEAD, CH_PER_TILE):
    wait_g(ch % NRING)
    fire_s(lgb, ch % GROUP, ch % NRING)
    wait_s(ch % NRING)
  wait_s(0)
  wait_idx()
  plsc.subcore_barrier()

  base = s * OUT_ROWS_PER_TILE
  pltpu.sync_copy(
      agg_sh.at[pl.ds(base, OUT_ROWS_PER_TILE)],
      out.at[pl.ds(base, OUT_ROWS_PER_TILE), pl.ds(c * HALF, HALF)])
  pltpu.sync_copy(
      agg_sh.at[pl.ds(REGION + base, OUT_ROWS_PER_TILE)],
      out.at[pl.ds(base, OUT_ROWS_PER_TILE), pl.ds(HID + c * HALF, HALF)])


_seg_sum = functools.partial(
    pl.kernel,
    out_type=jax.ShapeDtypeStruct((OUT_ROWS, 2 * HID), jnp.float32),
    mesh=plsc.VectorSubcoreMesh(core_axis_name="c", subcore_axis_name="s"),
    scratch_types=[
        pltpu.VMEM((2, GROUP, CHUNK), jnp.int32),
        pltpu.VMEM((2, GROUP, CHUNK), jnp.int32),
        pltpu.VMEM((NRING, CHUNK, HALF), jnp.float32),
        pltpu.VMEM_SHARED((AGG_ROWS, HALF), jnp.float32),
        pltpu.SemaphoreType.DMA,
        pltpu.SemaphoreType.DMA,
        pltpu.SemaphoreType.DMA,
    ],
    compiler_params=pltpu.CompilerParams(use_tc_tiling_on_sc=False),
)(_seg_sum_body)


def _mlp_body(x3, w1, b1, w2, b2, w3, b3, w4, b4, out):
  xx = jnp.concatenate([x3[0], x3[1]], axis=0)
  h = jnp.maximum(jnp.dot(xx, w1[...], preferred_element_type=jnp.float32)
                  + b1[...], 0.0)
  h = jnp.maximum(jnp.dot(h, w2[...], preferred_element_type=jnp.float32)
                  + b2[...], 0.0)
  h = jnp.maximum(jnp.dot(h, w3[...], preferred_element_type=jnp.float32)
                  + b3[...], 0.0) + h
  h = jnp.maximum(jnp.dot(h, w4[...], preferred_element_type=jnp.float32)
                  + b4[...], 0.0) + h
  out[...] = jnp.concatenate([h[:ROW_BLK], h[ROW_BLK:]], axis=1)


def _full(shape):
  return pl.BlockSpec(shape, lambda i: (0,) * len(shape))


def _mlp(x3, w1t, b1, w2t, b2, w3t, b3, w4t, b4):
  return pl.pallas_call(
      _mlp_body,
      grid=(GRID,),
      in_specs=[
          pl.BlockSpec((2, ROW_BLK, IN_DIM), lambda i: (0, i, 0)),
          _full((IN_DIM, HID)), _full((1, HID)),
          _full((HID, HID)), _full((1, HID)),
          _full((HID, HID)), _full((1, HID)),
          _full((HID, HID)), _full((1, HID)),
      ],
      out_specs=pl.BlockSpec((ROW_BLK, 2 * HID), lambda i: (i, 0)),
      out_shape=jax.ShapeDtypeStruct((NHALF, 2 * HID), jnp.float32),
  )(x3, w1t, b1, w2t, b2, w3t, b3, w4t, b4)


def _conv_combine_body(aggp, hp, wr, br, wroot, out):
  g = jnp.dot(aggp[...], wr[...], preferred_element_type=jnp.float32)
  g += jnp.dot(hp[...], wroot[...], preferred_element_type=jnp.float32)
  out[...] = jnp.maximum(g + br[...], 0.0)


def _conv_combine(aggp, hp, wr_bd, br_bd, wroot_bd):
  return pl.pallas_call(
      _conv_combine_body,
      grid=(GRID,),
      in_specs=[
          pl.BlockSpec((ROW_BLK, 2 * HID), lambda i: (i, 0)),
          pl.BlockSpec((ROW_BLK, 2 * HID), lambda i: (i, 0)),
          _full((2 * HID, 2 * HID)), _full((1, 2 * HID)),
          _full((2 * HID, 2 * HID)),
      ],
      out_specs=pl.BlockSpec((ROW_BLK, 2 * HID), lambda i: (i, 0)),
      out_shape=jax.ShapeDtypeStruct((NHALF, 2 * HID), jnp.float32),
  )(aggp, hp, wr_bd, br_bd, wroot_bd)


def _final_body(aggp, hp, wr, br, wroot, w5, b5, out):
  g = jnp.dot(aggp[...], wr[...], preferred_element_type=jnp.float32)
  g += jnp.dot(hp[...], wroot[...], preferred_element_type=jnp.float32)
  h2 = jnp.maximum(g + br[...], 0.0)
  out[...] = jnp.dot(h2, w5[...], preferred_element_type=jnp.float32) + b5[...]


def _final(aggp, hp, wr_bd, br_bd, wroot_bd, w5_bd, b5_bd):
  return pl.pallas_call(
      _final_body,
      grid=(GRID,),
      in_specs=[
          pl.BlockSpec((ROW_BLK, 2 * HID), lambda i: (i, 0)),
          pl.BlockSpec((ROW_BLK, 2 * HID), lambda i: (i, 0)),
          _full((2 * HID, 2 * HID)), _full((1, 2 * HID)),
          _full((2 * HID, 2 * HID)),
          _full((2 * HID, 2 * OUT_DIM)), _full((1, 2 * OUT_DIM)),
      ],
      out_specs=pl.BlockSpec((ROW_BLK, 2 * OUT_DIM), lambda i: (i, 0)),
      out_shape=jax.ShapeDtypeStruct((NHALF, 2 * OUT_DIM), jnp.float32),
  )(aggp, hp, wr_bd, br_bd, wroot_bd, w5_bd, b5_bd)


def _blockdiag(a):
  n, m = a.shape
  z = jnp.zeros((n, m), a.dtype)
  return jnp.concatenate([
      jnp.concatenate([a, z], axis=1),
      jnp.concatenate([z, a], axis=1)], axis=0)


def kernel(x, edge_index, W1, b1, W2, b2, W3, b3, W4, b4,
           Wrel1, brel1, Wroot1, Wrel2, brel2, Wroot2, W5, b5):
  src = edge_index[0].astype(jnp.int32)
  dst = edge_index[1].astype(jnp.int32)

  npad = E_PAD - N_EDGES
  i_pad = jnp.arange(npad, dtype=jnp.int32)
  srcp = jnp.concatenate([src, i_pad % N_NODES])
  src4 = jnp.where(srcp < NHALF, 4 * srcp, 4 * (srcp - NHALF) + 2)
  idx2 = jnp.stack([src4, src4 + 1]).reshape(NC, NCHUNKS, CHUNK)
  dst2 = jnp.where(dst < NHALF, dst, dst + TRASH)
  pad_dst = jnp.where(i_pad % 2 == 0,
                      NHALF + (i_pad // 2) % TRASH,
                      REGION + NHALF + (i_pad // 2) % TRASH)
  dstc = jnp.concatenate([dst2, pad_dst]).reshape(NCHUNKS, CHUNK)
  zfull = jnp.zeros((AGG_ROWS, HALF), jnp.float32)

  x3 = x.reshape(2, NHALF, IN_DIM)
  hp = _mlp(x3, W1.T, b1.reshape(1, HID), W2.T, b2.reshape(1, HID),
            W3.T, b3.reshape(1, HID), W4.T, b4.reshape(1, HID))

  wrel1_bd = _blockdiag(Wrel1.T)
  wroot1_bd = _blockdiag(Wroot1.T)
  brel1_bd = jnp.concatenate([brel1, brel1]).reshape(1, 2 * HID)
  wrel2_bd = _blockdiag(Wrel2.T)
  wroot2_bd = _blockdiag(Wroot2.T)
  brel2_bd = jnp.concatenate([brel2, brel2]).reshape(1, 2 * HID)
  w5_bd = _blockdiag(W5.T)
  b5_bd = jnp.concatenate([b5, b5]).reshape(1, 2 * OUT_DIM)

  agg1 = _seg_sum(hp.reshape(4 * NHALF, HALF), idx2, dstc, zfull)
  h1p = _conv_combine(agg1, hp, wrel1_bd, brel1_bd, wroot1_bd)

  agg2 = _seg_sum(h1p.reshape(4 * NHALF, HALF), idx2, dstc, zfull)
  outp = _final(agg2, h1p, wrel2_bd, brel2_bd, wroot2_bd,
                w5_bd, b5_bd)
  return jnp.concatenate([outp[:, :OUT_DIM], outp[:, OUT_DIM:]], axis=0)
